# baseline (speedup 1.0000x reference)
import functools

import jax
import jax.numpy as jnp
from jax import lax
from jax.experimental import pallas as pl
from jax.experimental.pallas import tpu as pltpu
from jax.experimental.pallas import tpu_sc as plsc

B = 16
T = 32768
F = 128
NC = 2
NS = 16
NW = NC * NS
T_SC = 12288
ROWS_PER_W = T_SC // NW
CH = 128
NCH = ROWS_PER_W // CH
TBLK = 2048
N_TC = T - T_SC


def _seg_pool_body(feat_hbm, ids_hbm, sums_hbm,
                   ids_v, feat_v0, feat_v1, feat_v2, zf_v, acc_sh,
                   gsem0, gsem1, gsem2, ssem0, ssem1, ssem2):
    cid = lax.axis_index("c")
    sid = lax.axis_index("s")
    wid = cid * NS + sid
    base = wid * ROWS_PER_W

    bufs = (feat_v0, feat_v1, feat_v2)
    gsems = (gsem0, gsem1, gsem2)
    ssems = (ssem0, ssem1, ssem2)

    pltpu.sync_copy(ids_hbm.at[wid], ids_v)
    g_desc = [
        pltpu.async_copy(feat_hbm.at[pl.ds(base + j * CH, CH)], bufs[j], gsems[j])
        for j in range(NCH)
    ]

    off = sid * B
    for j in range(NCH):
        for k in range(CH // 16):
            sl = pl.ds(k * 16, 16)
            ids_v[j, sl] = ids_v[j, sl] + off

    zero = jnp.zeros((16,), dtype=jnp.float32)
    for i in range(B):
        for j in range(F // 16):
            zf_v[i, pl.ds(j * 16, 16)] = zero
    pltpu.sync_copy(zf_v, acc_sh.at[pl.ds(sid * B, B)])

    s_desc = []
    for j in range(NCH):
        g_desc[j].wait()
        s_desc.append(pltpu.async_copy(
            bufs[j], acc_sh.at[ids_v.at[j]], ssems[j], add=True))
    for d in s_desc:
        d.wait()

    pltpu.sync_copy(acc_sh.at[pl.ds(sid * B, B)], sums_hbm.at[wid])


_seg_pool = pl.kernel(
    _seg_pool_body,
    out_type=jax.ShapeDtypeStruct((NW, B, F), jnp.float32),
    mesh=plsc.VectorSubcoreMesh(core_axis_name="c", subcore_axis_name="s"),
    scratch_types=[
        pltpu.VMEM((NCH, CH), jnp.int32),
        pltpu.VMEM((CH, F), jnp.float32),
        pltpu.VMEM((CH, F), jnp.float32),
        pltpu.VMEM((CH, F), jnp.float32),
        pltpu.VMEM((B, F), jnp.float32),
        pltpu.VMEM_SHARED((NS * B, F), jnp.float32),
        pltpu.SemaphoreType.DMA,
        pltpu.SemaphoreType.DMA,
        pltpu.SemaphoreType.DMA,
        pltpu.SemaphoreType.DMA,
        pltpu.SemaphoreType.DMA,
        pltpu.SemaphoreType.DMA,
    ],
)


def _counts_body(ids_ref, cnt_ref):
    ids = ids_ref[...]
    cols = [jnp.sum((ids == b).astype(jnp.float32)).reshape(1, 1)
            for b in range(B)]
    cnt_ref[...] = jnp.concatenate(cols, axis=1)


_counts = pl.pallas_call(
    _counts_body,
    out_shape=jax.ShapeDtypeStruct((1, B), jnp.float32),
)


def _tcsum_body(ids_ref, feat_ref, acc_ref):
    i = pl.program_id(0)

    @pl.when(i == 0)
    def _z():
        acc_ref[...] = jnp.zeros_like(acc_ref)

    ids = ids_ref[...]
    feat = feat_ref[...]
    oh = (ids == lax.broadcasted_iota(jnp.int32, (1, B), 1)
          ).astype(jnp.float32)
    acc_ref[...] += lax.dot_general(
        oh, feat, (((0,), (0,)), ((), ())),
        preferred_element_type=jnp.float32,
        precision=lax.Precision.HIGHEST)


_tcsum = pl.pallas_call(
    _tcsum_body,
    grid=(N_TC // TBLK,),
    in_specs=[
        pl.BlockSpec((TBLK, 1), lambda i: (i + T_SC // TBLK, 0)),
        pl.BlockSpec((TBLK, F), lambda i: (i + T_SC // TBLK, 0)),
    ],
    out_specs=pl.BlockSpec((B, F), lambda i: (0, 0)),
    out_shape=jax.ShapeDtypeStruct((B, F), jnp.float32),
)


def _combine_body(sums_ref, tc_ref, cnt_ref, out_ref):
    s = tc_ref[...]
    for w in range(NW):
        s = s + sums_ref[w]
    c = cnt_ref[0, :][:, None]
    out_ref[...] = s / jnp.maximum(c, 1.0)


_combine = pl.pallas_call(
    _combine_body,
    out_shape=jax.ShapeDtypeStruct((B, F), jnp.float32),
)


@jax.jit
def kernel(flat_points, flat_features, segment_ids):
    ids32 = segment_ids.astype(jnp.int32)
    ids2d = ids32.reshape(T // CH, CH)
    ids_col = ids32.reshape(T, 1)
    ids_sc = ids32[:T_SC].reshape(NW, NCH, CH)
    sums = _seg_pool(flat_features, ids_sc)
    cnt = _counts(ids2d)
    tc_part = _tcsum(ids_col, flat_features)
    pooled = _combine(sums, tc_part, cnt)
    return (flat_points, pooled)


# speedup vs baseline: 1.3917x; 1.3917x over previous
import functools

import jax
import jax.numpy as jnp
from jax import lax
from jax.experimental import pallas as pl
from jax.experimental.pallas import tpu as pltpu
from jax.experimental.pallas import tpu_sc as plsc

B = 16
T = 32768
F = 128
NC = 2
NS = 16
NW = NC * NS
T_SC = 8192
ROWS_PER_W = T_SC // NW
CH = 128
NCH = ROWS_PER_W // CH
TBLK = 2048
N_TC = T - T_SC


def _seg_pool_body(feat_hbm, ids_hbm, sums_hbm,
                   ids_v, feat_v0, feat_v1, zf_v, acc_sh,
                   gsem0, gsem1, ssem0, ssem1):
    cid = lax.axis_index("c")
    sid = lax.axis_index("s")
    wid = cid * NS + sid
    base = wid * ROWS_PER_W

    bufs = (feat_v0, feat_v1)
    gsems = (gsem0, gsem1)
    ssems = (ssem0, ssem1)

    g_desc = [
        pltpu.async_copy(feat_hbm.at[pl.ds(base + j * CH, CH)], bufs[j], gsems[j])
        for j in range(NCH)
    ]
    for j in range(NCH):
        pltpu.sync_copy(ids_hbm.at[pl.ds(base + j * CH, CH)], ids_v.at[j])

    off = sid * B
    for j in range(NCH):
        for k in range(CH // 16):
            sl = pl.ds(k * 16, 16)
            ids_v[j, sl] = ids_v[j, sl] + off

    zero = jnp.zeros((16,), dtype=jnp.float32)
    for i in range(B):
        for j in range(F // 16):
            zf_v[i, pl.ds(j * 16, 16)] = zero
    pltpu.sync_copy(zf_v, acc_sh.at[pl.ds(sid * B, B)])

    s_desc = []
    for j in range(NCH):
        g_desc[j].wait()
        s_desc.append(pltpu.async_copy(
            bufs[j], acc_sh.at[ids_v.at[j]], ssems[j], add=True))
    for d in s_desc:
        d.wait()

    pltpu.sync_copy(acc_sh.at[pl.ds(sid * B, B)], sums_hbm.at[wid])


_seg_pool = pl.kernel(
    _seg_pool_body,
    out_type=jax.ShapeDtypeStruct((NW, B, F), jnp.float32),
    mesh=plsc.VectorSubcoreMesh(core_axis_name="c", subcore_axis_name="s"),
    scratch_types=[
        pltpu.VMEM((NCH, CH), jnp.int32),
        pltpu.VMEM((CH, F), jnp.float32),
        pltpu.VMEM((CH, F), jnp.float32),
        pltpu.VMEM((B, F), jnp.float32),
        pltpu.VMEM_SHARED((NS * B, F), jnp.float32),
        pltpu.SemaphoreType.DMA,
        pltpu.SemaphoreType.DMA,
        pltpu.SemaphoreType.DMA,
        pltpu.SemaphoreType.DMA,
    ],
)


def _counts_body(ids_ref, cnt_ref, lo_ref, hi_ref):
    ids = ids_ref[...]
    counts = [jnp.sum((ids == b).astype(jnp.int32)).reshape(1, 1)
              for b in range(B)]
    cnt_ref[...] = jnp.concatenate(
        [c.astype(jnp.float32) for c in counts], axis=1)
    los, his = [], []
    running = jnp.zeros((1, 1), jnp.int32)
    for b in range(B):
        los.append(running)
        running = running + counts[b]
        his.append(running)
    lo_ref[...] = jnp.concatenate(los, axis=1)
    hi_ref[...] = jnp.concatenate(his, axis=1)


_counts = pl.pallas_call(
    _counts_body,
    out_shape=(
        jax.ShapeDtypeStruct((1, B), jnp.float32),
        jax.ShapeDtypeStruct((1, B), jnp.int32),
        jax.ShapeDtypeStruct((1, B), jnp.int32),
    ),
)


def _tcsum_body(lo_ref, hi_ref, feat_ref, acc_ref):
    i = pl.program_id(0)

    @pl.when(i == 0)
    def _z():
        acc_ref[...] = jnp.zeros_like(acc_ref)

    rowtok = (T_SC + i * TBLK
              + lax.broadcasted_iota(jnp.int32, (TBLK, B), 0))
    lo = lo_ref[...]
    hi = hi_ref[...]
    oh = ((rowtok >= lo) & (rowtok < hi)).astype(jnp.float32)
    feat = feat_ref[...]
    f_hi = feat.astype(jnp.bfloat16).astype(jnp.float32)
    f_lo = feat - f_hi
    dims = (((0,), (0,)), ((), ()))
    acc_ref[...] += (
        lax.dot_general(oh, f_hi, dims, preferred_element_type=jnp.float32)
        + lax.dot_general(oh, f_lo, dims, preferred_element_type=jnp.float32))


_tcsum = pl.pallas_call(
    _tcsum_body,
    grid=(N_TC // TBLK,),
    in_specs=[
        pl.BlockSpec((1, B), lambda i: (0, 0)),
        pl.BlockSpec((1, B), lambda i: (0, 0)),
        pl.BlockSpec((TBLK, F), lambda i: (i + T_SC // TBLK, 0)),
    ],
    out_specs=pl.BlockSpec((B, F), lambda i: (0, 0)),
    out_shape=jax.ShapeDtypeStruct((B, F), jnp.float32),
)


def _combine_body(sums_ref, tc_ref, cnt_ref, out_ref):
    s = tc_ref[...]
    for w in range(NW):
        s = s + sums_ref[w]
    c = cnt_ref[0, :][:, None]
    out_ref[...] = s / jnp.maximum(c, 1.0)


_combine = pl.pallas_call(
    _combine_body,
    out_shape=jax.ShapeDtypeStruct((B, F), jnp.float32),
)


@jax.jit
def kernel(flat_points, flat_features, segment_ids):
    ids32 = segment_ids.astype(jnp.int32)
    ids2d = ids32.reshape(T // CH, CH)
    sums = _seg_pool(flat_features, ids32)
    cnt, lo, hi = _counts(ids2d)
    tc_part = _tcsum(lo, hi, flat_features)
    pooled = _combine(sums, tc_part, cnt)
    return (flat_points, pooled)


# speedup vs baseline: 1.4267x; 1.0251x over previous
import functools

import jax
import jax.numpy as jnp
from jax import lax
from jax.experimental import pallas as pl
from jax.experimental.pallas import tpu as pltpu
from jax.experimental.pallas import tpu_sc as plsc

B = 16
T = 32768
F = 128
NC = 2
NS = 16
NW = NC * NS
T_SC = 16384
ROWS_PER_W = T_SC // NW
CH = 128
NCH = ROWS_PER_W // CH
TBLK = 4096
N_TC = T - T_SC


def _seg_pool_body(feat_hbm, ids_hbm, sums_hbm, *scratch):
    ids_v = scratch[0]
    bufs = scratch[1:1 + NCH]
    zf_v = scratch[1 + NCH]
    acc_sh = scratch[2 + NCH]
    gsems = scratch[3 + NCH:3 + 2 * NCH]
    ssems = scratch[3 + 2 * NCH:3 + 3 * NCH]

    cid = lax.axis_index("c")
    sid = lax.axis_index("s")
    wid = cid * NS + sid
    base = wid * ROWS_PER_W

    g_desc = [
        pltpu.async_copy(feat_hbm.at[pl.ds(base + j * CH, CH)], bufs[j], gsems[j])
        for j in range(NCH)
    ]
    for j in range(NCH):
        pltpu.sync_copy(ids_hbm.at[pl.ds(base + j * CH, CH)], ids_v.at[j])

    off = sid * B
    for j in range(NCH):
        for k in range(CH // 16):
            sl = pl.ds(k * 16, 16)
            ids_v[j, sl] = ids_v[j, sl] + off

    zero = jnp.zeros((16,), dtype=jnp.float32)
    for i in range(B):
        for j in range(F // 16):
            zf_v[i, pl.ds(j * 16, 16)] = zero
    pltpu.sync_copy(zf_v, acc_sh.at[pl.ds(sid * B, B)])

    s_desc = []
    for j in range(NCH):
        g_desc[j].wait()
        s_desc.append(pltpu.async_copy(
            bufs[j], acc_sh.at[ids_v.at[j]], ssems[j], add=True))
    for d in s_desc:
        d.wait()

    pltpu.sync_copy(acc_sh.at[pl.ds(sid * B, B)], sums_hbm.at[wid])


_seg_pool = pl.kernel(
    _seg_pool_body,
    out_type=jax.ShapeDtypeStruct((NW, B, F), jnp.float32),
    mesh=plsc.VectorSubcoreMesh(core_axis_name="c", subcore_axis_name="s"),
    scratch_types=(
        [pltpu.VMEM((NCH, CH), jnp.int32)]
        + [pltpu.VMEM((CH, F), jnp.float32)] * NCH
        + [pltpu.VMEM((B, F), jnp.float32)]
        + [pltpu.VMEM_SHARED((NS * B, F), jnp.float32)]
        + [pltpu.SemaphoreType.DMA] * (2 * NCH)
    ),
)


def _counts_body(ids_ref, cnt_ref, lo_ref, hi_ref):
    ids = ids_ref[...]
    counts = [jnp.sum((ids == b).astype(jnp.int32)).reshape(1, 1)
              for b in range(B)]
    cnt_ref[...] = jnp.concatenate(
        [c.astype(jnp.float32) for c in counts], axis=1)
    los, his = [], []
    running = jnp.zeros((1, 1), jnp.int32)
    for b in range(B):
        los.append(running)
        running = running + counts[b]
        his.append(running)
    lo_ref[...] = jnp.concatenate(los, axis=1)
    hi_ref[...] = jnp.concatenate(his, axis=1)


_counts = pl.pallas_call(
    _counts_body,
    out_shape=(
        jax.ShapeDtypeStruct((1, B), jnp.float32),
        jax.ShapeDtypeStruct((1, B), jnp.int32),
        jax.ShapeDtypeStruct((1, B), jnp.int32),
    ),
)


def _tcsum_body(lo_ref, hi_ref, feat_ref, acc_ref):
    i = pl.program_id(0)

    @pl.when(i == 0)
    def _z():
        acc_ref[...] = jnp.zeros_like(acc_ref)

    rowtok = (T_SC + i * TBLK
              + lax.broadcasted_iota(jnp.int32, (TBLK, B), 0))
    lo = lo_ref[...]
    hi = hi_ref[...]
    oh = ((rowtok >= lo) & (rowtok < hi)).astype(jnp.float32)
    feat = feat_ref[...]
    f_hi = feat.astype(jnp.bfloat16).astype(jnp.float32)
    f_lo = feat - f_hi
    dims = (((0,), (0,)), ((), ()))
    acc_ref[...] += (
        lax.dot_general(oh, f_hi, dims, preferred_element_type=jnp.float32)
        + lax.dot_general(oh, f_lo, dims, preferred_element_type=jnp.float32))


_tcsum = pl.pallas_call(
    _tcsum_body,
    grid=(N_TC // TBLK,),
    in_specs=[
        pl.BlockSpec((1, B), lambda i: (0, 0)),
        pl.BlockSpec((1, B), lambda i: (0, 0)),
        pl.BlockSpec((TBLK, F), lambda i: (i + T_SC // TBLK, 0)),
    ],
    out_specs=pl.BlockSpec((B, F), lambda i: (0, 0)),
    out_shape=jax.ShapeDtypeStruct((B, F), jnp.float32),
)


def _combine_body(sums_ref, tc_ref, cnt_ref, out_ref):
    s = tc_ref[...]
    for w in range(NW):
        s = s + sums_ref[w]
    c = cnt_ref[0, :][:, None]
    out_ref[...] = s / jnp.maximum(c, 1.0)


_combine = pl.pallas_call(
    _combine_body,
    out_shape=jax.ShapeDtypeStruct((B, F), jnp.float32),
)


@jax.jit
def kernel(flat_points, flat_features, segment_ids):
    ids32 = segment_ids.astype(jnp.int32)
    ids2d = ids32.reshape(T // CH, CH)
    sums = _seg_pool(flat_features, ids32)
    cnt, lo, hi = _counts(ids2d)
    tc_part = _tcsum(lo, hi, flat_features)
    pooled = _combine(sums, tc_part, cnt)
    return (flat_points, pooled)


# speedup vs baseline: 1.5343x; 1.0754x over previous
import functools

import jax
import jax.numpy as jnp
from jax import lax
from jax.experimental import pallas as pl
from jax.experimental.pallas import tpu as pltpu
from jax.experimental.pallas import tpu_sc as plsc

B = 16
T = 32768
F = 128
NC = 2
NS = 16
NW = NC * NS
T_SC = 12288
ROWS_PER_W = T_SC // NW
CH = 128
NCH = ROWS_PER_W // CH
TBLK = 4096
N_TC = T - T_SC


def _seg_pool_body(feat_hbm, ids_hbm, sums_hbm, *scratch):
    ids_v = scratch[0]
    bufs = scratch[1:1 + NCH]
    zf_v = scratch[1 + NCH]
    acc_sh = scratch[2 + NCH]
    gsems = scratch[3 + NCH:3 + 2 * NCH]
    ssems = scratch[3 + 2 * NCH:3 + 3 * NCH]

    cid = lax.axis_index("c")
    sid = lax.axis_index("s")
    wid = cid * NS + sid
    base = wid * ROWS_PER_W

    g_desc = [
        pltpu.async_copy(feat_hbm.at[pl.ds(base + j * CH, CH)], bufs[j], gsems[j])
        for j in range(NCH)
    ]
    for j in range(NCH):
        pltpu.sync_copy(ids_hbm.at[pl.ds(base + j * CH, CH)], ids_v.at[j])

    off = sid * B
    for j in range(NCH):
        for k in range(CH // 16):
            sl = pl.ds(k * 16, 16)
            ids_v[j, sl] = ids_v[j, sl] + off

    zero = jnp.zeros((16,), dtype=jnp.float32)
    for i in range(B):
        for j in range(F // 16):
            zf_v[i, pl.ds(j * 16, 16)] = zero
    pltpu.sync_copy(zf_v, acc_sh.at[pl.ds(sid * B, B)])

    s_desc = []
    for j in range(NCH):
        g_desc[j].wait()
        s_desc.append(pltpu.async_copy(
            bufs[j], acc_sh.at[ids_v.at[j]], ssems[j], add=True))
    for d in s_desc:
        d.wait()

    pltpu.sync_copy(acc_sh.at[pl.ds(sid * B, B)], sums_hbm.at[wid])


_seg_pool = pl.kernel(
    _seg_pool_body,
    out_type=jax.ShapeDtypeStruct((NW, B, F), jnp.float32),
    mesh=plsc.VectorSubcoreMesh(core_axis_name="c", subcore_axis_name="s"),
    scratch_types=(
        [pltpu.VMEM((NCH, CH), jnp.int32)]
        + [pltpu.VMEM((CH, F), jnp.float32)] * NCH
        + [pltpu.VMEM((B, F), jnp.float32)]
        + [pltpu.VMEM_SHARED((NS * B, F), jnp.float32)]
        + [pltpu.SemaphoreType.DMA] * (2 * NCH)
    ),
)


def _counts_body(ids_ref, cnt_ref, lo_ref, hi_ref):
    ids = ids_ref[...]
    counts = [jnp.sum((ids == b).astype(jnp.int32)).reshape(1, 1)
              for b in range(B)]
    cnt_ref[...] = jnp.concatenate(
        [c.astype(jnp.float32) for c in counts], axis=1)
    los, his = [], []
    running = jnp.zeros((1, 1), jnp.int32)
    for b in range(B):
        los.append(running)
        running = running + counts[b]
        his.append(running)
    lo_ref[...] = jnp.concatenate(los, axis=1)
    hi_ref[...] = jnp.concatenate(his, axis=1)


_counts = pl.pallas_call(
    _counts_body,
    out_shape=(
        jax.ShapeDtypeStruct((1, B), jnp.float32),
        jax.ShapeDtypeStruct((1, B), jnp.int32),
        jax.ShapeDtypeStruct((1, B), jnp.int32),
    ),
)


def _tcsum_body(lo_ref, hi_ref, feat_ref, acc_ref):
    i = pl.program_id(0)

    @pl.when(i == 0)
    def _z():
        acc_ref[...] = jnp.zeros_like(acc_ref)

    rowtok = (T_SC + i * TBLK
              + lax.broadcasted_iota(jnp.int32, (TBLK, B), 0))
    lo = lo_ref[...]
    hi = hi_ref[...]
    oh = ((rowtok >= lo) & (rowtok < hi)).astype(jnp.float32)
    feat = feat_ref[...]
    f_hi = feat.astype(jnp.bfloat16).astype(jnp.float32)
    f_lo = feat - f_hi
    dims = (((0,), (0,)), ((), ()))
    acc_ref[...] += (
        lax.dot_general(oh, f_hi, dims, preferred_element_type=jnp.float32)
        + lax.dot_general(oh, f_lo, dims, preferred_element_type=jnp.float32))


_tcsum = pl.pallas_call(
    _tcsum_body,
    grid=(N_TC // TBLK,),
    in_specs=[
        pl.BlockSpec((1, B), lambda i: (0, 0)),
        pl.BlockSpec((1, B), lambda i: (0, 0)),
        pl.BlockSpec((TBLK, F), lambda i: (i + T_SC // TBLK, 0)),
    ],
    out_specs=pl.BlockSpec((B, F), lambda i: (0, 0)),
    out_shape=jax.ShapeDtypeStruct((B, F), jnp.float32),
)


def _combine_body(sums_ref, tc_ref, cnt_ref, out_ref):
    s = tc_ref[...]
    for w in range(NW):
        s = s + sums_ref[w]
    c = cnt_ref[0, :][:, None]
    out_ref[...] = s / jnp.maximum(c, 1.0)


_combine = pl.pallas_call(
    _combine_body,
    out_shape=jax.ShapeDtypeStruct((B, F), jnp.float32),
)


@jax.jit
def kernel(flat_points, flat_features, segment_ids):
    ids32 = segment_ids.astype(jnp.int32)
    ids2d = ids32.reshape(T // CH, CH)
    sums = _seg_pool(flat_features, ids32)
    cnt, lo, hi = _counts(ids2d)
    tc_part = _tcsum(lo, hi, flat_features)
    pooled = _combine(sums, tc_part, cnt)
    return (flat_points, pooled)
